# baseline (speedup 1.0000x reference)
import jax
import jax.numpy as jnp
from jax.experimental import pallas as pl
from jax.experimental.pallas import tpu as pltpu

_NQ = 196
_D = 1536
_N = 100000
_T = 4000
_NT = _N // _T
_K = 9
_INF = float("inf")

_SORT8 = ((0, 1), (2, 3), (4, 5), (6, 7),
          (0, 2), (1, 3), (4, 6), (5, 7),
          (1, 2), (5, 6),
          (0, 4), (1, 5), (2, 6), (3, 7),
          (2, 4), (3, 5),
          (1, 2), (3, 4), (5, 6))

_MERGE8 = ((0, 4), (1, 5), (2, 6), (3, 7),
           (0, 2), (1, 3), (4, 6), (5, 7),
           (0, 1), (2, 3), (4, 5), (6, 7))


def _cmp(v, a, b):
    lo = jnp.minimum(v[a], v[b])
    hi = jnp.maximum(v[a], v[b])
    v[a], v[b] = lo, hi


def _body(f_ref, b_ref, o_ref, fbf_ref, run_ref):
    i = pl.program_id(0)

    @pl.when(i == 0)
    def _init():
        run_ref[...] = jnp.full((_K, _NQ, 128), _INF, jnp.float32)
        fbf_ref[...] = (-2.0 * f_ref[...]).astype(jnp.bfloat16)

    f = fbf_ref[...]
    b = b_ref[...]
    dn = (((1,), (1,)), ((), ()))
    bsq = (b * b).astype(jnp.bfloat16)
    bn8 = jax.lax.dot_general(jnp.ones((8, _D), jnp.bfloat16), bsq, dn,
                              preferred_element_type=jnp.float32)
    mm = jax.lax.dot_general(
        f, b.astype(jnp.bfloat16), dn,
        preferred_element_type=jnp.float32)
    d2 = mm + bn8[0:1]

    runs = [run_ref[j] for j in range(_K)]
    nfull = _T // 128
    rag = jnp.concatenate(
        [d2[:, nfull * 128:_T],
         jnp.full((_NQ, 128 - (_T - nfull * 128)), _INF, jnp.float32)],
        axis=1)
    chunks = [d2[:, c * 128:(c + 1) * 128] for c in range(nfull)] + [rag]
    for g in range(len(chunks) // 8):
        s = chunks[g * 8:(g + 1) * 8]
        for a, b_ in _SORT8:
            _cmp(s, a, b_)
        ell = [runs[0]] + [jnp.minimum(runs[j], s[8 - j])
                           for j in range(1, _K)]
        _cmp(ell, 0, 8)
        u = ell[1:]
        for a, b_ in _MERGE8:
            _cmp(u, a, b_)
        runs = [ell[0]] + u
    for j in range(_K):
        run_ref[j] = runs[j]

    @pl.when(i == pl.num_programs(0) - 1)
    def _fin():
        fr = f_ref[...]
        fn = jnp.sum(fr * fr, axis=1, keepdims=True)
        cand = jnp.concatenate([run_ref[j] for j in range(_K)], axis=1)
        ii = jax.lax.broadcasted_iota(jnp.int32, cand.shape, 1)
        total = jnp.zeros((_NQ, 1), jnp.float32)
        for _ in range(_K):
            m = jnp.min(cand, axis=1, keepdims=True)
            total = total + jnp.sqrt(jnp.maximum(m + fn, 1e-12))
            hit = cand == m
            first = jnp.min(jnp.where(hit, ii, jnp.int32(1 << 30)),
                            axis=1, keepdims=True)
            cand = jnp.where(ii == first, _INF, cand)
        o_ref[...] = jnp.broadcast_to(total / float(_K), (_NQ, 128))


def kernel(features, memory_bank):
    out = pl.pallas_call(
        _body,
        grid=(_NT,),
        in_specs=[
            pl.BlockSpec((_NQ, _D), lambda i: (0, 0)),
            pl.BlockSpec((_T, _D), lambda i: (i, 0)),
        ],
        out_specs=pl.BlockSpec((_NQ, 128), lambda i: (0, 0)),
        out_shape=jax.ShapeDtypeStruct((_NQ, 128), jnp.float32),
        scratch_shapes=[
            pltpu.VMEM((_NQ, _D), jnp.bfloat16),
            pltpu.VMEM((_K, _NQ, 128), jnp.float32),
        ],
        compiler_params=pltpu.CompilerParams(
            dimension_semantics=("arbitrary",)),
    )(features.astype(jnp.float32), memory_bank)
    return out[:, 0]


# speedup vs baseline: 1.0591x; 1.0591x over previous
import jax
import jax.numpy as jnp
from jax.experimental import pallas as pl
from jax.experimental.pallas import tpu as pltpu

_NQ = 196
_D = 1536
_N = 100000
_T = 4000
_NT = _N // _T
_K = 9
_INF = float("inf")

_SORT8 = ((0, 1), (2, 3), (4, 5), (6, 7),
          (0, 2), (1, 3), (4, 6), (5, 7),
          (1, 2), (5, 6),
          (0, 4), (1, 5), (2, 6), (3, 7),
          (2, 4), (3, 5),
          (1, 2), (3, 4), (5, 6))

_MERGE8 = ((0, 4), (1, 5), (2, 6), (3, 7),
           (0, 2), (1, 3), (4, 6), (5, 7),
           (0, 1), (2, 3), (4, 5), (6, 7))


def _cmp(v, a, b):
    lo = jnp.minimum(v[a], v[b])
    hi = jnp.maximum(v[a], v[b])
    v[a], v[b] = lo, hi


def _body(f_ref, b_ref, o_ref, fbf_ref, run_ref):
    i = pl.program_id(0)

    @pl.when(i == 0)
    def _init():
        run_ref[...] = jnp.full((_K, _NQ, 128), _INF, jnp.float32)
        fbf_ref[...] = (-2.0 * f_ref[...]).astype(jnp.bfloat16)

    f = fbf_ref[...]
    b = b_ref[...]
    bn = jnp.sum(b * b, axis=1)
    mm = jax.lax.dot_general(
        f, b.astype(jnp.bfloat16),
        dimension_numbers=(((1,), (1,)), ((), ())),
        preferred_element_type=jnp.float32)
    d2 = mm + bn[None, :]

    runs = [run_ref[j] for j in range(_K)]
    nfull = _T // 128
    rag = jnp.concatenate(
        [d2[:, nfull * 128:_T],
         jnp.full((_NQ, 128 - (_T - nfull * 128)), _INF, jnp.float32)],
        axis=1)
    chunks = [d2[:, c * 128:(c + 1) * 128] for c in range(nfull)] + [rag]
    for g in range(len(chunks) // 8):
        s = chunks[g * 8:(g + 1) * 8]
        for a, b_ in _SORT8:
            _cmp(s, a, b_)
        ell = [runs[0]] + [jnp.minimum(runs[j], s[8 - j])
                           for j in range(1, _K)]
        _cmp(ell, 0, 8)
        u = ell[1:]
        for a, b_ in _MERGE8:
            _cmp(u, a, b_)
        runs = [ell[0]] + u
    for j in range(_K):
        run_ref[j] = runs[j]

    @pl.when(i == pl.num_programs(0) - 1)
    def _fin():
        fr = f_ref[...]
        fn = jnp.sum(fr * fr, axis=1, keepdims=True)
        cand = jnp.concatenate([run_ref[j] for j in range(_K)], axis=1)
        ii = jax.lax.broadcasted_iota(jnp.int32, cand.shape, 1)
        total = jnp.zeros((_NQ, 1), jnp.float32)
        for _ in range(_K):
            m = jnp.min(cand, axis=1, keepdims=True)
            total = total + jnp.sqrt(jnp.maximum(m + fn, 1e-12))
            hit = cand == m
            first = jnp.min(jnp.where(hit, ii, jnp.int32(1 << 30)),
                            axis=1, keepdims=True)
            cand = jnp.where(ii == first, _INF, cand)
        o_ref[...] = jnp.broadcast_to(total / float(_K), (_NQ, 128))


def kernel(features, memory_bank):
    out = pl.pallas_call(
        _body,
        grid=(_NT,),
        in_specs=[
            pl.BlockSpec((_NQ, _D), lambda i: (0, 0)),
            pl.BlockSpec((_T, _D), lambda i: (i, 0)),
        ],
        out_specs=pl.BlockSpec((_NQ, 128), lambda i: (0, 0)),
        out_shape=jax.ShapeDtypeStruct((_NQ, 128), jnp.float32),
        scratch_shapes=[
            pltpu.VMEM((_NQ, _D), jnp.bfloat16),
            pltpu.VMEM((_K, _NQ, 128), jnp.float32),
        ],
        compiler_params=pltpu.CompilerParams(
            dimension_semantics=("arbitrary",)),
    )(features.astype(jnp.float32), memory_bank)
    return out[:, 0]
